# baseline (speedup 1.0000x reference)
import functools
import math

import jax
import jax.numpy as jnp
from jax import lax
from jax.experimental import pallas as pl
from jax.experimental.pallas import tpu as pltpu
from jax.experimental.pallas import tpu_sc as plsc


def _backbone_body(x_ref, wb_ref, bb_ref, wg_ref, bg_ref, fbf_ref, sct_ref):
    f = jnp.dot(x_ref[...], wb_ref[...], preferred_element_type=jnp.float32)
    f = f + bb_ref[...]
    fu = lax.bitcast_convert_type(f, jnp.uint32)
    rb = (fu + jnp.uint32(0x7FFF) +
          ((fu >> jnp.uint32(16)) & jnp.uint32(1))) >> jnp.uint32(16)
    dh = f.shape[1] // 2
    word = rb[:, :dh] | (rb[:, dh:] << jnp.uint32(16))
    fbf_ref[...] = lax.bitcast_convert_type(word, jnp.int32)
    sct_ref[...] = lax.dot_general(
        wg_ref[...], f, (((0,), (1,)), ((), ())),
        preferred_element_type=jnp.float32) + bg_ref[...]


def _backbone(x, Wb, bb, Wg, bg):
    B, D = x.shape
    E = Wg.shape[1]
    BM = min(512, B)
    return pl.pallas_call(
        _backbone_body,
        grid=(B // BM,),
        in_specs=[
            pl.BlockSpec((BM, D), lambda i: (i, 0)),
            pl.BlockSpec((D, D), lambda i: (0, 0)),
            pl.BlockSpec((1, D), lambda i: (0, 0)),
            pl.BlockSpec((D, E), lambda i: (0, 0)),
            pl.BlockSpec((E, 1), lambda i: (0, 0)),
        ],
        out_specs=[
            pl.BlockSpec((BM, D // 2), lambda i: (i, 0)),
            pl.BlockSpec((E, BM), lambda i: (0, i)),
        ],
        out_shape=[
            jax.ShapeDtypeStruct((B, D // 2), jnp.int32),
            jax.ShapeDtypeStruct((E, B), jnp.float32),
        ],
    )(x, Wb, bb.reshape(1, D), Wg, bg.reshape(E, 1))


def _select_body(M, sct_ref, selt_ref, minv_ref, ws_ref):
    s = sct_ref[...]
    E, B = s.shape
    bits = lax.bitcast_convert_type(s, jnp.int32)
    key = jnp.where(bits < 0, bits ^ jnp.int32(0x7FFFFFFF), bits)
    ukey = lax.bitcast_convert_type(key, jnp.uint32) ^ jnp.uint32(0x80000000)
    u_hi = (ukey >> jnp.uint32(16)).astype(jnp.int32)
    u_lo = (ukey & jnp.uint32(0xFFFF)).astype(jnp.int32)

    def bsearch(cnt_ge, target):
        def step(_, lohi):
            lo, hi = lohi
            mid = (lo + hi) // 2
            ok = cnt_ge(mid) >= target
            return jnp.where(ok, mid, lo), jnp.where(ok, hi, mid)
        lo0 = jnp.zeros((E, 1), jnp.int32)
        hi0 = jnp.full((E, 1), 65536, jnp.int32)
        lo, _ = lax.fori_loop(0, 16, step, (lo0, hi0))
        return lo

    tm = jnp.int32(M)
    cnt_hi = lambda v: jnp.sum((u_hi >= v).astype(jnp.int32), axis=1, keepdims=True)
    hstar = bsearch(cnt_hi, tm)
    n_gt_hi = jnp.sum((u_hi > hstar).astype(jnp.int32), axis=1, keepdims=True)
    r = tm - n_gt_hi
    eq_hi = u_hi == hstar
    cnt_lo = lambda v: jnp.sum((eq_hi & (u_lo >= v)).astype(jnp.int32), axis=1,
                               keepdims=True)
    lstar = bsearch(cnt_lo, r)

    gt = (u_hi > hstar) | (eq_hi & (u_lo > lstar))
    tie = eq_hi & (u_lo == lstar)
    need = tm - jnp.sum(gt.astype(jnp.int32), axis=1, keepdims=True)
    tok = lax.broadcasted_iota(jnp.int32, (E, B), 1)

    def tstep(_, lohi):
        lo, hi = lohi
        mid = (lo + hi) // 2
        cnt = jnp.sum((tie & (tok <= mid)).astype(jnp.int32), axis=1,
                      keepdims=True)
        ok = cnt >= need
        return jnp.where(ok, lo, mid), jnp.where(ok, mid, hi)

    nbits = max(1, (B - 1).bit_length())
    lo0 = jnp.full((E, 1), -1, jnp.int32)
    hi0 = jnp.full((E, 1), B - 1, jnp.int32)
    _, vstar = lax.fori_loop(0, nbits, tstep, (lo0, hi0))
    sel = gt | (tie & (tok <= vstar))

    m = jnp.sum(sel.astype(jnp.float32), axis=0, keepdims=True)
    minv_ref[...] = 1.0 / jnp.maximum(m, 1.0)
    selt_ref[...] = sel.astype(jnp.int32)

    seli = sel.astype(jnp.int32)
    nblk = B // 256
    cols = []
    for rblk in range(nblk):
        if rblk == 0:
            sb = jnp.zeros((E, 1), jnp.int32)
        else:
            sb = jnp.sum(seli * (tok < rblk * 256), axis=1, keepdims=True)
        cols.append(jnp.minimum(sb >> 7, (M - 384) // 128))
    ws_ref[...] = jnp.concatenate(cols, axis=1)


def _select(scoresT, M):
    E, B = scoresT.shape
    return pl.pallas_call(
        functools.partial(_select_body, M),
        out_shape=[
            jax.ShapeDtypeStruct((E, B), jnp.int32),
            jax.ShapeDtypeStruct((1, B), jnp.float32),
            jax.ShapeDtypeStruct((E, B // 256), jnp.int32),
        ],
    )(scoresT)


def _mlp_body(nh, mb, feat_ref, w1_ref, b1_ref, w2_ref, b2_ref, wcol_ref,
              out_ref, acc_ref):
    hblk, m = pl.program_id(1), pl.program_id(2)
    wds = lax.bitcast_convert_type(feat_ref[...], jnp.uint32)
    left = lax.bitcast_convert_type(wds << jnp.uint32(16),
                                    jnp.float32).astype(jnp.bfloat16)
    right = lax.bitcast_convert_type(wds & jnp.uint32(0xFFFF0000),
                                     jnp.float32).astype(jnp.bfloat16)
    f = jnp.concatenate([left, right], axis=1)
    w1 = w1_ref[0].astype(jnp.bfloat16)
    hpre = jnp.dot(f, w1, preferred_element_type=jnp.float32)
    hpre = hpre + b1_ref[0]
    hr = jnp.maximum(hpre, 0.0).astype(jnp.bfloat16)
    w2 = w2_ref[0].astype(jnp.bfloat16)
    part = jnp.dot(hr, w2, preferred_element_type=jnp.float32)
    asl = acc_ref.at[pl.ds(m * mb, mb)]

    @pl.when(hblk == 0)
    def _():
        asl[...] = part + b2_ref[0]

    @pl.when(hblk > 0)
    def _():
        asl[...] = asl[...] + part

    @pl.when(hblk == nh - 1)
    def _():
        out_ref[...] = (asl[...] * wcol_ref[...]).astype(out_ref.dtype)


def _expert_mlp(feat_sel, W1, b1, W2, b2, w_flat, out_dtype=jnp.bfloat16):
    E, D, H = W1.shape
    O = W2.shape[2]
    M = feat_sel.shape[0] // E
    HB = min(1024, H)
    NH = H // HB
    NM = 2 if M >= 1024 else 1
    MB = M // NM
    return pl.pallas_call(
        functools.partial(_mlp_body, NH, MB),
        grid=(E, NH, NM),
        in_specs=[
            pl.BlockSpec((MB, D // 2), lambda e, h, m: (e * NM + m, 0)),
            pl.BlockSpec((1, D, HB), lambda e, h, m: (e, 0, h)),
            pl.BlockSpec((1, 1, HB), lambda e, h, m: (e, 0, h)),
            pl.BlockSpec((1, HB, O), lambda e, h, m: (e, h, 0)),
            pl.BlockSpec((1, 1, O), lambda e, h, m: (e, 0, 0)),
            pl.BlockSpec((MB, 1), lambda e, h, m: (e * NM + m, 0)),
        ],
        out_specs=pl.BlockSpec((MB, O), lambda e, h, m: (e * NM + m, 0)),
        out_shape=jax.ShapeDtypeStruct((E * M, O), out_dtype),
        scratch_shapes=[pltpu.VMEM((M, O), jnp.float32)],
    )(feat_sel, W1, b1.reshape(E, 1, H),
      W2, b2.reshape(E, 1, O), w_flat.reshape(E * M, 1))


def _compact_sc(selT, minv_flat, M):
    E, B = selT.shape
    info = plsc.get_sparse_core_info()
    NC = info.num_cores
    L = 16
    CH = 2048
    NCH = B // CH
    mesh = plsc.VectorSubcoreMesh(core_axis_name="c", subcore_axis_name="s")

    @functools.partial(
        pl.kernel, mesh=mesh,
        out_type=[
            jax.ShapeDtypeStruct((E * M,), jnp.int32),
            jax.ShapeDtypeStruct((E * M,), jnp.float32),
        ],
        scratch_types=[
            pltpu.VMEM((CH,), jnp.int32),
            pltpu.VMEM((CH,), jnp.float32),
            pltpu.VMEM((M,), jnp.int32),
            pltpu.VMEM((M,), jnp.float32),
            pltpu.VMEM((16,), jnp.int32),
        ],
        compiler_params=pltpu.CompilerParams(needs_layout_passes=False),
    )
    def k(sel_hbm, minv_hbm, idx_hbm, w_hbm, sel_v, minv_v, idxo_v, wo_v,
          off_v):
        wid = lax.axis_index("s") * NC + lax.axis_index("c")

        @pl.when(wid < E)
        def _():
            off_v[...] = jnp.zeros((L,), jnp.int32)

            def chunk(c, _):
                pltpu.sync_copy(sel_hbm.at[wid, pl.ds(c * CH, CH)], sel_v)
                pltpu.sync_copy(minv_hbm.at[pl.ds(c * CH, CH)], minv_v)
                lane = lax.iota(jnp.int32, L)
                off = off_v[...]
                for i in range(CH // L):
                    vec = sel_v[pl.ds(i * L, L)]
                    mask = vec > 0
                    cum = plsc.cumsum(vec)
                    pos = cum - 1 + off
                    toks = lane + (c * CH + i * L)
                    plsc.store_scatter(idxo_v, [pos], toks, mask=mask)
                    plsc.store_scatter(wo_v, [pos], minv_v[pl.ds(i * L, L)],
                                       mask=mask)
                    off = off + plsc.all_reduce_population_count(mask)
                off_v[...] = off
                return 0

            lax.fori_loop(0, NCH, chunk, 0)
            pltpu.sync_copy(idxo_v, idx_hbm.at[pl.ds(wid * M, M)])
            pltpu.sync_copy(wo_v, w_hbm.at[pl.ds(wid * M, M)])

    return k(selT, minv_flat)


def _gather_sc(table, idx_flat):
    N, Dm = table.shape
    P = idx_flat.shape[0]
    info = plsc.get_sparse_core_info()
    NC, NS = info.num_cores, info.num_subcores
    NW = NC * NS
    per_w = P // NW
    CH = 32
    NCH = per_w // CH
    mesh = plsc.VectorSubcoreMesh(core_axis_name="c", subcore_axis_name="s")

    @functools.partial(
        pl.kernel, mesh=mesh,
        out_type=jax.ShapeDtypeStruct((P, Dm), table.dtype),
        scratch_types=[
            pltpu.VMEM((CH,), jnp.int32),
            pltpu.VMEM((CH, Dm), table.dtype),
            pltpu.SemaphoreType.DMA,
        ],
        compiler_params=pltpu.CompilerParams(needs_layout_passes=False),
    )
    def k(tab_hbm, idx_hbm, out_hbm, idx_v, rows_v, sem):
        wid = lax.axis_index("s") * NC + lax.axis_index("c")
        base = wid * per_w

        def chunk(c, _):
            b = base + c * CH
            pltpu.sync_copy(idx_hbm.at[pl.ds(b, CH)], idx_v)
            pltpu.async_copy(tab_hbm.at[idx_v], rows_v, sem).wait()
            pltpu.sync_copy(rows_v, out_hbm.at[pl.ds(b, CH)])
            return 0

        lax.fori_loop(0, NCH, chunk, 0)

    return k(table, idx_flat)


def _combine_body(tblk, ws_ref, i0, i1, i2, w0, w1, w2, out_ref):
    t, e = pl.program_id(0), pl.program_id(1)
    base = t * tblk
    idxw = jnp.concatenate([i0[0], i1[0], i2[0]], axis=1)
    wyw = jnp.concatenate([w0[...], w1[...], w2[...]], axis=0)
    ii = lax.broadcasted_iota(jnp.int32, (tblk, 384), 0) + base
    St = (ii == idxw).astype(jnp.bfloat16)
    part = jnp.dot(St, wyw, preferred_element_type=jnp.float32)

    @pl.when(e == 0)
    def _():
        out_ref[...] = part

    @pl.when(e > 0)
    def _():
        out_ref[...] = out_ref[...] + part


def _combine(wy, idx3, ws, B):
    E, _, M = idx3.shape
    O = wy.shape[1]
    TBLK = 256
    MB = M // 128

    def idx_spec(k):
        return pl.BlockSpec(
            (1, 1, 128), lambda t, e, ws_ref: (e, 0, ws_ref[e, t] + k))

    def wy_spec(k):
        return pl.BlockSpec(
            (128, O), lambda t, e, ws_ref: (e * MB + ws_ref[e, t] + k, 0))

    grid_spec = pltpu.PrefetchScalarGridSpec(
        num_scalar_prefetch=1,
        grid=(B // TBLK, E),
        in_specs=[idx_spec(0), idx_spec(1), idx_spec(2),
                  wy_spec(0), wy_spec(1), wy_spec(2)],
        out_specs=pl.BlockSpec((TBLK, O), lambda t, e, ws_ref: (t, 0)),
    )
    return pl.pallas_call(
        functools.partial(_combine_body, TBLK),
        grid_spec=grid_spec,
        out_shape=jax.ShapeDtypeStruct((B, O), jnp.float32),
    )(ws, idx3, idx3, idx3, wy, wy, wy)


def kernel(x, Wb, bb, Wg, bg, W1, b1, W2, b2):
    B, D = x.shape
    E = Wg.shape[1]
    M = max(1, int(math.ceil(B / float(E))))

    features, scoresT = _backbone(x, Wb, bb, Wg, bg)
    selT, minv, ws = _select(scoresT, M)
    idx_flat, w_flat = _compact_sc(selT, minv.reshape(-1), M)
    feat_sel = _gather_sc(features, idx_flat)
    wy = _expert_mlp(feat_sel, W1, b1, W2, b2, w_flat)
    combined = _combine(wy, idx_flat.reshape(E, 1, M), ws, B)
    return combined


# speedup vs baseline: 1.2556x; 1.2556x over previous
import functools
import math

import jax
import jax.numpy as jnp
from jax import lax
from jax.experimental import pallas as pl
from jax.experimental.pallas import tpu as pltpu
from jax.experimental.pallas import tpu_sc as plsc


def _backbone_body(x_ref, wb_ref, bb_ref, wg_ref, bg_ref, fbf_ref, sct_ref):
    f = jnp.dot(x_ref[...], wb_ref[...], preferred_element_type=jnp.float32)
    f = f + bb_ref[...]
    fu = lax.bitcast_convert_type(f, jnp.uint32)
    rb = (fu + jnp.uint32(0x7FFF) +
          ((fu >> jnp.uint32(16)) & jnp.uint32(1))) >> jnp.uint32(16)
    dh = f.shape[1] // 2
    word = rb[:, :dh] | (rb[:, dh:] << jnp.uint32(16))
    fbf_ref[...] = lax.bitcast_convert_type(word, jnp.int32)
    sct_ref[...] = lax.dot_general(
        wg_ref[...], f, (((0,), (1,)), ((), ())),
        preferred_element_type=jnp.float32) + bg_ref[...]


def _backbone(x, Wb, bb, Wg, bg):
    B, D = x.shape
    E = Wg.shape[1]
    BM = min(512, B)
    return pl.pallas_call(
        _backbone_body,
        grid=(B // BM,),
        in_specs=[
            pl.BlockSpec((BM, D), lambda i: (i, 0)),
            pl.BlockSpec((D, D), lambda i: (0, 0)),
            pl.BlockSpec((1, D), lambda i: (0, 0)),
            pl.BlockSpec((D, E), lambda i: (0, 0)),
            pl.BlockSpec((E, 1), lambda i: (0, 0)),
        ],
        out_specs=[
            pl.BlockSpec((BM, D // 2), lambda i: (i, 0)),
            pl.BlockSpec((E, BM), lambda i: (0, i)),
        ],
        out_shape=[
            jax.ShapeDtypeStruct((B, D // 2), jnp.int32),
            jax.ShapeDtypeStruct((E, B), jnp.float32),
        ],
    )(x, Wb, bb.reshape(1, D), Wg, bg.reshape(E, 1))


def _select_body(M, sct_ref, selt_ref, minv_ref, ws_ref):
    s = sct_ref[...]
    E, B = s.shape
    bits = lax.bitcast_convert_type(s, jnp.int32)
    key = jnp.where(bits < 0, bits ^ jnp.int32(0x7FFFFFFF), bits)
    ukey = lax.bitcast_convert_type(key, jnp.uint32) ^ jnp.uint32(0x80000000)
    u_hi = (ukey >> jnp.uint32(16)).astype(jnp.int32)
    u_lo = (ukey & jnp.uint32(0xFFFF)).astype(jnp.int32)

    def bsearch(cnt_ge, target):
        def step(_, lohi):
            lo, hi = lohi
            mid = (lo + hi) // 2
            ok = cnt_ge(mid) >= target
            return jnp.where(ok, mid, lo), jnp.where(ok, hi, mid)
        lo0 = jnp.zeros((E, 1), jnp.int32)
        hi0 = jnp.full((E, 1), 65536, jnp.int32)
        lo, _ = lax.fori_loop(0, 16, step, (lo0, hi0))
        return lo

    tm = jnp.int32(M)
    cnt_hi = lambda v: jnp.sum((u_hi >= v).astype(jnp.int32), axis=1, keepdims=True)
    hstar = bsearch(cnt_hi, tm)
    n_gt_hi = jnp.sum((u_hi > hstar).astype(jnp.int32), axis=1, keepdims=True)
    r = tm - n_gt_hi
    eq_hi = u_hi == hstar
    cnt_lo = lambda v: jnp.sum((eq_hi & (u_lo >= v)).astype(jnp.int32), axis=1,
                               keepdims=True)
    lstar = bsearch(cnt_lo, r)

    gt = (u_hi > hstar) | (eq_hi & (u_lo > lstar))
    tie = eq_hi & (u_lo == lstar)
    need = tm - jnp.sum(gt.astype(jnp.int32), axis=1, keepdims=True)
    tok = lax.broadcasted_iota(jnp.int32, (E, B), 1)

    def tstep(_, lohi):
        lo, hi = lohi
        mid = (lo + hi) // 2
        cnt = jnp.sum((tie & (tok <= mid)).astype(jnp.int32), axis=1,
                      keepdims=True)
        ok = cnt >= need
        return jnp.where(ok, lo, mid), jnp.where(ok, mid, hi)

    nbits = max(1, (B - 1).bit_length())
    lo0 = jnp.full((E, 1), -1, jnp.int32)
    hi0 = jnp.full((E, 1), B - 1, jnp.int32)
    _, vstar = lax.fori_loop(0, nbits, tstep, (lo0, hi0))
    sel = gt | (tie & (tok <= vstar))

    m = jnp.sum(sel.astype(jnp.float32), axis=0, keepdims=True)
    minv_ref[...] = 1.0 / jnp.maximum(m, 1.0)
    selt_ref[...] = sel.astype(jnp.int32)

    seli = sel.astype(jnp.int32)
    nblk = B // 256
    cols = []
    for rblk in range(nblk):
        if rblk == 0:
            sb = jnp.zeros((E, 1), jnp.int32)
        else:
            sb = jnp.sum(seli * (tok < rblk * 256), axis=1, keepdims=True)
        cols.append(jnp.minimum(sb >> 7, (M - 384) // 128))
    ws_ref[...] = jnp.concatenate(cols, axis=1)


def _select(scoresT, M):
    E, B = scoresT.shape
    return pl.pallas_call(
        functools.partial(_select_body, M),
        out_shape=[
            jax.ShapeDtypeStruct((E, B), jnp.int32),
            jax.ShapeDtypeStruct((1, B), jnp.float32),
            jax.ShapeDtypeStruct((E, B // 256), jnp.int32),
        ],
    )(scoresT)


def _mlp_body(nh, mb, feat_ref, w1_ref, b1_ref, w2_ref, b2_ref, wcol_ref,
              out_ref, acc_ref):
    hblk, m = pl.program_id(1), pl.program_id(2)
    wds = lax.bitcast_convert_type(feat_ref[...], jnp.uint32)
    left = lax.bitcast_convert_type(wds << jnp.uint32(16),
                                    jnp.float32).astype(jnp.bfloat16)
    right = lax.bitcast_convert_type(wds & jnp.uint32(0xFFFF0000),
                                     jnp.float32).astype(jnp.bfloat16)
    f = jnp.concatenate([left, right], axis=1)
    w1 = w1_ref[0].astype(jnp.bfloat16)
    hpre = jnp.dot(f, w1, preferred_element_type=jnp.float32)
    hpre = hpre + b1_ref[0]
    hr = jnp.maximum(hpre, 0.0).astype(jnp.bfloat16)
    w2 = w2_ref[0].astype(jnp.bfloat16)
    part = jnp.dot(hr, w2, preferred_element_type=jnp.float32)
    asl = acc_ref.at[pl.ds(m * mb, mb)]

    @pl.when(hblk == 0)
    def _():
        asl[...] = part + b2_ref[0]

    @pl.when(hblk > 0)
    def _():
        asl[...] = asl[...] + part

    @pl.when(hblk == nh - 1)
    def _():
        out_ref[...] = (asl[...] * wcol_ref[...]).astype(out_ref.dtype)


def _expert_mlp(feat_sel, W1, b1, W2, b2, w_flat, out_dtype=jnp.bfloat16):
    E, D, H = W1.shape
    O = W2.shape[2]
    M = feat_sel.shape[0] // E
    HB = min(1024, H)
    NH = H // HB
    NM = 2 if M >= 1024 else 1
    MB = M // NM
    return pl.pallas_call(
        functools.partial(_mlp_body, NH, MB),
        grid=(E, NH, NM),
        in_specs=[
            pl.BlockSpec((MB, D // 2), lambda e, h, m: (e * NM + m, 0)),
            pl.BlockSpec((1, D, HB), lambda e, h, m: (e, 0, h)),
            pl.BlockSpec((1, 1, HB), lambda e, h, m: (e, 0, h)),
            pl.BlockSpec((1, HB, O), lambda e, h, m: (e, h, 0)),
            pl.BlockSpec((1, 1, O), lambda e, h, m: (e, 0, 0)),
            pl.BlockSpec((MB, 1), lambda e, h, m: (e * NM + m, 0)),
        ],
        out_specs=pl.BlockSpec((MB, O), lambda e, h, m: (e * NM + m, 0)),
        out_shape=jax.ShapeDtypeStruct((E * M, O), out_dtype),
        scratch_shapes=[pltpu.VMEM((M, O), jnp.float32)],
    )(feat_sel, W1, b1.reshape(E, 1, H),
      W2, b2.reshape(E, 1, O), w_flat.reshape(E * M, 1))


def _compact_sc(selT, minv_flat, M):
    E, B = selT.shape
    info = plsc.get_sparse_core_info()
    NC = info.num_cores
    L = 16
    CH = 2048
    NCH = B // CH
    mesh = plsc.VectorSubcoreMesh(core_axis_name="c", subcore_axis_name="s")

    @functools.partial(
        pl.kernel, mesh=mesh,
        out_type=[
            jax.ShapeDtypeStruct((E * M,), jnp.int32),
            jax.ShapeDtypeStruct((E * M,), jnp.float32),
        ],
        scratch_types=[
            pltpu.VMEM((CH,), jnp.int32),
            pltpu.VMEM((CH,), jnp.float32),
            pltpu.VMEM((M,), jnp.int32),
            pltpu.VMEM((M,), jnp.float32),
            pltpu.VMEM((16,), jnp.int32),
        ],
        compiler_params=pltpu.CompilerParams(needs_layout_passes=False),
    )
    def k(sel_hbm, minv_hbm, idx_hbm, w_hbm, sel_v, minv_v, idxo_v, wo_v,
          off_v):
        wid = lax.axis_index("s") * NC + lax.axis_index("c")

        @pl.when(wid < E)
        def _():
            off_v[...] = jnp.zeros((L,), jnp.int32)

            def chunk(c, _):
                pltpu.sync_copy(sel_hbm.at[wid, pl.ds(c * CH, CH)], sel_v)
                pltpu.sync_copy(minv_hbm.at[pl.ds(c * CH, CH)], minv_v)
                lane = lax.iota(jnp.int32, L)
                off = off_v[...]
                for i in range(CH // L):
                    vec = sel_v[pl.ds(i * L, L)]
                    mask = vec > 0
                    cum = plsc.cumsum(vec)
                    pos = cum - 1 + off
                    toks = lane + (c * CH + i * L)
                    plsc.store_scatter(idxo_v, [pos], toks, mask=mask)
                    plsc.store_scatter(wo_v, [pos], minv_v[pl.ds(i * L, L)],
                                       mask=mask)
                    off = off + plsc.all_reduce_population_count(mask)
                off_v[...] = off
                return 0

            lax.fori_loop(0, NCH, chunk, 0)
            pltpu.sync_copy(idxo_v, idx_hbm.at[pl.ds(wid * M, M)])
            pltpu.sync_copy(wo_v, w_hbm.at[pl.ds(wid * M, M)])

    return k(selT, minv_flat)


def _gather_sc(table, idx_flat):
    N, Dm = table.shape
    P = idx_flat.shape[0]
    info = plsc.get_sparse_core_info()
    NC, NS = info.num_cores, info.num_subcores
    NW = NC * NS
    per_w = P // NW
    CH = 32
    NCH = per_w // CH
    mesh = plsc.VectorSubcoreMesh(core_axis_name="c", subcore_axis_name="s")

    @functools.partial(
        pl.kernel, mesh=mesh,
        out_type=jax.ShapeDtypeStruct((P, Dm), table.dtype),
        scratch_types=[
            pltpu.VMEM((CH,), jnp.int32),
            pltpu.VMEM((CH, Dm), table.dtype),
            pltpu.SemaphoreType.DMA,
        ],
        compiler_params=pltpu.CompilerParams(needs_layout_passes=False),
    )
    def k(tab_hbm, idx_hbm, out_hbm, idx_v, rows_v, sem):
        wid = lax.axis_index("s") * NC + lax.axis_index("c")
        base = wid * per_w

        def chunk(c, _):
            b = base + c * CH
            pltpu.sync_copy(idx_hbm.at[pl.ds(b, CH)], idx_v)
            pltpu.async_copy(tab_hbm.at[idx_v], rows_v, sem).wait()
            pltpu.sync_copy(rows_v, out_hbm.at[pl.ds(b, CH)])
            return 0

        lax.fori_loop(0, NCH, chunk, 0)

    return k(table, idx_flat)


def _combine_body(tblk, ne, ws_ref, *refs):
    t = pl.program_id(0)
    base = t * tblk
    idx_refs = refs[:3 * ne]
    wy_refs = refs[3 * ne:6 * ne]
    out_ref = refs[6 * ne]
    ii = lax.broadcasted_iota(jnp.int32, (tblk, 384), 0) + base
    parts_idx = []
    for e in range(ne):
        idxw = jnp.concatenate(
            [idx_refs[3 * e + k][0] for k in range(3)], axis=1)
        parts_idx.append((ii == idxw).astype(jnp.bfloat16))
    St = jnp.concatenate(parts_idx, axis=1)
    wyw = jnp.concatenate([w[...] for w in wy_refs], axis=0)
    out_ref[...] = jnp.dot(St, wyw, preferred_element_type=jnp.float32)


def _combine(wy, idx3, ws, B):
    E, _, M = idx3.shape
    O = wy.shape[1]
    TBLK = 256
    MB = M // 128

    def idx_spec(e, k):
        return pl.BlockSpec(
            (1, 1, 128), lambda t, ws_ref: (e, 0, ws_ref[e, t] + k))

    def wy_spec(e, k):
        return pl.BlockSpec(
            (128, O), lambda t, ws_ref: (e * MB + ws_ref[e, t] + k, 0))

    grid_spec = pltpu.PrefetchScalarGridSpec(
        num_scalar_prefetch=1,
        grid=(B // TBLK,),
        in_specs=([idx_spec(e, k) for e in range(E) for k in range(3)]
                  + [wy_spec(e, k) for e in range(E) for k in range(3)]),
        out_specs=pl.BlockSpec((TBLK, O), lambda t, ws_ref: (t, 0)),
    )
    return pl.pallas_call(
        functools.partial(_combine_body, TBLK, E),
        grid_spec=grid_spec,
        out_shape=jax.ShapeDtypeStruct((B, O), jnp.float32),
    )(ws, *([idx3] * (3 * E)), *([wy] * (3 * E)))


def kernel(x, Wb, bb, Wg, bg, W1, b1, W2, b2):
    B, D = x.shape
    E = Wg.shape[1]
    M = max(1, int(math.ceil(B / float(E))))

    features, scoresT = _backbone(x, Wb, bb, Wg, bg)
    selT, minv, ws = _select(scoresT, M)
    idx_flat, w_flat = _compact_sc(selT, minv.reshape(-1), M)
    feat_sel = _gather_sc(features, idx_flat)
    wy = _expert_mlp(feat_sel, W1, b1, W2, b2, w_flat)
    combined = _combine(wy, idx_flat.reshape(E, 1, M), ws, B)
    return combined
